# baseline (speedup 1.0000x reference)
import functools

import jax
import jax.numpy as jnp
from jax import lax
from jax.experimental import pallas as pl
from jax.experimental.pallas import tpu as pltpu
from jax.experimental.pallas import tpu_sc as plsc

B = 256
N = 32768
C = 16
L = 16
NC = 2
NS = 16
NW = NC * NS
RPW = B // NW
K = 4096
NCHUNK = N // K
HIST = 256
GPC = K // L


def _body(values_hbm, labels_hbm, out_hbm,
          vbuf, lbuf, acc, cnt_acc, outb, cntb, cnt_all, rsbuf, clb_v,
          cnt_sh, sem_v0, sem_v1, sem_l0, sem_l1):
  cid = lax.axis_index("c")
  sid = lax.axis_index("s")
  wid = sid * NC + cid
  row_base = wid * RPW
  sem_v = (sem_v0, sem_v1)
  sem_l = (sem_l0, sem_l1)

  zeros = jnp.zeros((L,), jnp.float32)
  ones = jnp.ones((L,), jnp.float32)
  iota = lax.broadcasted_iota(jnp.int32, (L,), 0)
  lane_base = iota * C

  def _zero(i, _):
    acc[pl.ds(i * L, L)] = zeros
    return 0
  lax.fori_loop(0, (RPW * HIST) // L, _zero, 0)

  def _zero_cnt(i, _):
    cnt_acc[pl.ds(i * L, L)] = zeros
    return 0
  lax.fori_loop(0, HIST // L, _zero_cnt, 0)

  def copies(g, b):
    cv = pltpu.make_async_copy(
        values_hbm.at[pl.ds(row_base, RPW), pl.ds(g * K, K)],
        vbuf.at[b], sem_v[b])
    cl = pltpu.make_async_copy(
        labels_hbm.at[pl.ds(g * K, K)], lbuf.at[b], sem_l[b])
    return cv, cl

  for b in range(2):
    for c in copies(b, b):
      c.start()

  def _pair(git, _):
    for b in range(2):
      g = git * 2 + b
      for c in copies(g, b):
        c.wait()

      @pl.when(g + 2 < NCHUNK)
      def _():
        for c in copies(g + 2, b):
          c.start()

      @plsc.parallel_loop(0, GPC, unroll=4)
      def _group(j):
        lab = lbuf[b, pl.ds(j * L, L)]
        idx0 = lab + lane_base
        vs = [vbuf[b, r, pl.ds(j * L, L)] for r in range(RPW)]
        idxs = [idx0 + (r * HIST) for r in range(RPW)]
        for r in range(RPW):
          plsc.addupdate_scatter(acc, [idxs[r]], vs[r])

      def _cnt(jj, _):
        lab = lbuf[b, pl.ds((sid * (GPC // NS) + jj) * L, L)]
        plsc.addupdate_scatter(cnt_acc, [lab + lane_base], ones)
        return 0
      lax.fori_loop(0, GPC // NS, _cnt, 0)
    return 0
  lax.fori_loop(0, NCHUNK // 2, _pair, 0)

  def _redc(l, s):
    return s + cnt_acc[pl.ds(l * L, L)]
  cntb[...] = lax.fori_loop(0, L, _redc, zeros)
  pltpu.sync_copy(cntb, cnt_sh.at[sid])
  plsc.subcore_barrier()
  pltpu.sync_copy(cnt_sh, cnt_all)

  def _redall(t, s):
    return s + cnt_all[t, :]
  counts = lax.fori_loop(0, NS, _redall, zeros)

  present = counts > 0.5
  pos = plsc.cumsum(present.astype(jnp.int32)) - 1
  clb_v[...] = jnp.zeros((C,), jnp.int32)
  plsc.store_scatter(clb_v, [pos], iota, mask=present)
  clab = clb_v[...]

  for r in range(RPW):
    def _red(l, s):
      return s + acc[pl.ds(r * HIST + l * L, L)]
    rsbuf[...] = lax.fori_loop(0, L, _red, zeros)
    outb[r, :] = plsc.load_gather(rsbuf, [clab])
  pltpu.sync_copy(outb, out_hbm.at[pl.ds(row_base, RPW), :])


@jax.jit
def _pooled_sums(values, labels):
  return pl.kernel(
      _body,
      out_type=jax.ShapeDtypeStruct((B, C), jnp.float32),
      mesh=plsc.VectorSubcoreMesh(
          core_axis_name="c", subcore_axis_name="s",
          num_cores=NC, num_subcores=NS),
      compiler_params=pltpu.CompilerParams(needs_layout_passes=False),
      scratch_types=[
          pltpu.VMEM((2, RPW, K), jnp.float32),
          pltpu.VMEM((2, K), jnp.int32),
          pltpu.VMEM((RPW * HIST,), jnp.float32),
          pltpu.VMEM((HIST,), jnp.float32),
          pltpu.VMEM((RPW, C), jnp.float32),
          pltpu.VMEM((C,), jnp.float32),
          pltpu.VMEM((NS, C), jnp.float32),
          pltpu.VMEM((C,), jnp.float32),
          pltpu.VMEM((C,), jnp.int32),
          pltpu.VMEM_SHARED((NS, C), jnp.float32),
          pltpu.SemaphoreType.DMA,
          pltpu.SemaphoreType.DMA,
          pltpu.SemaphoreType.DMA,
          pltpu.SemaphoreType.DMA,
      ],
  )(values, labels)


def kernel(values, labels):
  return _pooled_sums(values, labels)


# speedup vs baseline: 1.2691x; 1.2691x over previous
import functools

import jax
import jax.numpy as jnp
from jax import lax
from jax.experimental import pallas as pl
from jax.experimental.pallas import tpu as pltpu
from jax.experimental.pallas import tpu_sc as plsc

B = 256
N = 32768
C = 16
L = 16
NC = 2
NS = 16
NW = NC * NS
R_SC = 64
RPW = R_SC // NW
K = 4096
NCHUNK = N // K
HIST = 256
GPC = K // L
R_TC = B - R_SC
RTCB = 64


def _body(values_hbm, labels_hbm, out_hbm,
          vbuf, lbuf, acc, cnt_acc, outb, cntb, cnt_all, rsbuf, clb_v,
          cnt_sh, sem_v0, sem_v1, sem_l0, sem_l1):
  cid = lax.axis_index("c")
  sid = lax.axis_index("s")
  wid = sid * NC + cid
  row_base = wid * RPW
  sem_v = (sem_v0, sem_v1)
  sem_l = (sem_l0, sem_l1)

  zeros = jnp.zeros((L,), jnp.float32)
  ones = jnp.ones((L,), jnp.float32)
  iota = lax.broadcasted_iota(jnp.int32, (L,), 0)
  lane_base = iota * C

  def _zero(i, _):
    acc[pl.ds(i * L, L)] = zeros
    return 0
  lax.fori_loop(0, (RPW * HIST) // L, _zero, 0)

  def _zero_cnt(i, _):
    cnt_acc[pl.ds(i * L, L)] = zeros
    return 0
  lax.fori_loop(0, HIST // L, _zero_cnt, 0)

  def start(g):
    b = g % 2
    cv = pltpu.async_copy(
        values_hbm.at[pl.ds(row_base, RPW), pl.ds(g * K, K)],
        vbuf.at[b], sem_v[b])
    cl = pltpu.async_copy(
        labels_hbm.at[pl.ds(g * K, K)], lbuf.at[b], sem_l[b])
    return cv, cl

  inflight = start(0)
  for g in range(NCHUNK):
    b = g % 2
    cv, cl = inflight
    cv.wait()
    cl.wait()
    if g + 1 < NCHUNK:
      inflight = start(g + 1)

    @plsc.parallel_loop(0, GPC, unroll=4)
    def _group(j):
      lab = lbuf[b, pl.ds(j * L, L)]
      idx0 = lab + lane_base
      vs = [vbuf[b, r, pl.ds(j * L, L)] for r in range(RPW)]
      idxs = [idx0 + (r * HIST) for r in range(RPW)]
      for r in range(RPW):
        plsc.addupdate_scatter(acc, [idxs[r]], vs[r])

    def _cnt(jj, _):
      lab = lbuf[b, pl.ds((sid * (GPC // NS) + jj) * L, L)]
      plsc.addupdate_scatter(cnt_acc, [lab + lane_base], ones)
      return 0
    lax.fori_loop(0, GPC // NS, _cnt, 0)

  def _redc(l, s):
    return s + cnt_acc[pl.ds(l * L, L)]
  cntb[...] = lax.fori_loop(0, L, _redc, zeros)
  pltpu.sync_copy(cntb, cnt_sh.at[sid])
  plsc.subcore_barrier()
  pltpu.sync_copy(cnt_sh, cnt_all)

  def _redall(t, s):
    return s + cnt_all[t, :]
  counts = lax.fori_loop(0, NS, _redall, zeros)

  present = counts > 0.5
  pos = plsc.cumsum(present.astype(jnp.int32)) - 1
  clb_v[...] = jnp.zeros((C,), jnp.int32)
  plsc.store_scatter(clb_v, [pos], iota, mask=present)
  clab = clb_v[...]

  for r in range(RPW):
    def _red(l, s):
      return s + acc[pl.ds(r * HIST + l * L, L)]
    rsbuf[...] = lax.fori_loop(0, L, _red, zeros)
    outb[r, :] = plsc.load_gather(rsbuf, [clab])
  pltpu.sync_copy(outb, out_hbm.at[pl.ds(row_base, RPW), :])


def _tc_body(lab_ref, val_ref, out_ref, acc_ref, cnt_ref):
  k = pl.program_id(1)

  @pl.when(k == 0)
  def _():
    acc_ref[...] = jnp.zeros_like(acc_ref)

  @pl.when(jnp.logical_and(pl.program_id(0) == 0, k == 0))
  def _():
    cnt_ref[...] = jnp.zeros_like(cnt_ref)

  lab = lab_ref[0, :]
  oh = (lab[:, None] == lax.broadcasted_iota(jnp.int32, (K, C), 1)
        ).astype(jnp.float32)
  acc_ref[...] += jnp.dot(val_ref[...], oh,
                          preferred_element_type=jnp.float32)

  @pl.when(pl.program_id(0) == 0)
  def _():
    oht = (lab[None, :] == lax.broadcasted_iota(jnp.int32, (C, K), 0)
           ).astype(jnp.float32)
    cnt_ref[...] += jnp.sum(oht, axis=1, keepdims=True)

  @pl.when(k == NCHUNK - 1)
  def _():
    presb = cnt_ref[...] > 0.5
    crow = lax.broadcasted_iota(jnp.int32, (C, C), 0)
    ccol = lax.broadcasted_iota(jnp.int32, (C, C), 1)
    tril = (ccol <= crow).astype(jnp.float32)
    posc = jnp.dot(tril, presb.astype(jnp.float32),
                   preferred_element_type=jnp.float32) - 1.0
    pmain = jnp.logical_and(presb, posc.astype(jnp.int32) == ccol)
    colhas = jnp.sum(pmain.astype(jnp.float32), axis=0, keepdims=True)
    pfill = jnp.logical_and(crow == 0, colhas < 0.5)
    perm = jnp.logical_or(pmain, pfill).astype(jnp.float32)
    out_ref[...] = jnp.dot(acc_ref[...], perm,
                           preferred_element_type=jnp.float32)


@jax.jit
def _pooled_sums(values, labels):
  sums_tc = pl.pallas_call(
      _tc_body,
      grid=(R_TC // RTCB, NCHUNK),
      in_specs=[
          pl.BlockSpec((1, K), lambda i, k: (0, k)),
          pl.BlockSpec((RTCB, K), lambda i, k: (i + R_SC // RTCB, k)),
      ],
      out_specs=pl.BlockSpec((RTCB, C), lambda i, k: (i, 0)),
      out_shape=jax.ShapeDtypeStruct((R_TC, C), jnp.float32),
      scratch_shapes=[
          pltpu.VMEM((RTCB, C), jnp.float32),
          pltpu.VMEM((C, 1), jnp.float32),
      ],
  )(labels.reshape(1, N), values)

  sums_sc = pl.kernel(
      _body,
      out_type=jax.ShapeDtypeStruct((R_SC, C), jnp.float32),
      mesh=plsc.VectorSubcoreMesh(
          core_axis_name="c", subcore_axis_name="s",
          num_cores=NC, num_subcores=NS),
      compiler_params=pltpu.CompilerParams(needs_layout_passes=False),
      scratch_types=[
          pltpu.VMEM((2, RPW, K), jnp.float32),
          pltpu.VMEM((2, K), jnp.int32),
          pltpu.VMEM((RPW * HIST,), jnp.float32),
          pltpu.VMEM((HIST,), jnp.float32),
          pltpu.VMEM((RPW, C), jnp.float32),
          pltpu.VMEM((C,), jnp.float32),
          pltpu.VMEM((NS, C), jnp.float32),
          pltpu.VMEM((C,), jnp.float32),
          pltpu.VMEM((C,), jnp.int32),
          pltpu.VMEM_SHARED((NS, C), jnp.float32),
          pltpu.SemaphoreType.DMA,
          pltpu.SemaphoreType.DMA,
          pltpu.SemaphoreType.DMA,
          pltpu.SemaphoreType.DMA,
      ],
  )(values, labels)

  return jnp.concatenate([sums_sc, sums_tc], axis=0)


def kernel(values, labels):
  return _pooled_sums(values, labels)


# speedup vs baseline: 1.5267x; 1.2030x over previous
import functools

import jax
import jax.numpy as jnp
from jax import lax
from jax.experimental import pallas as pl
from jax.experimental.pallas import tpu as pltpu
from jax.experimental.pallas import tpu_sc as plsc

B = 256
N = 32768
C = 16
L = 16
NC = 2
NS = 16
NW = NC * NS
R_SC = 64
RPW = R_SC // NW
K = 4096
NCHUNK = N // K
HIST = 256
GPC = K // L
R_TC = B - R_SC
NSEG = 8
WID = NSEG * C


def _body(values_hbm, labels_hbm, out_hbm,
          vbuf, lbuf, acc, cnt_acc, outb, cntb, cnt_all, rsbuf, clb_v,
          cnt_sh, sem_v0, sem_v1, sem_l0, sem_l1):
  cid = lax.axis_index("c")
  sid = lax.axis_index("s")
  wid = sid * NC + cid
  out_base = wid * RPW
  row_base = R_TC + out_base
  sem_v = (sem_v0, sem_v1)
  sem_l = (sem_l0, sem_l1)

  zeros = jnp.zeros((L,), jnp.float32)
  ones = jnp.ones((L,), jnp.float32)
  iota = lax.broadcasted_iota(jnp.int32, (L,), 0)
  lane_base = iota * C

  def _zero(i, _):
    acc[pl.ds(i * L, L)] = zeros
    return 0
  lax.fori_loop(0, (RPW * HIST) // L, _zero, 0)

  def _zero_cnt(i, _):
    cnt_acc[pl.ds(i * L, L)] = zeros
    return 0
  lax.fori_loop(0, HIST // L, _zero_cnt, 0)

  def start(g):
    b = g % 2
    cv = pltpu.async_copy(
        values_hbm.at[pl.ds(row_base, RPW), pl.ds(g * K, K)],
        vbuf.at[b], sem_v[b])
    cl = pltpu.async_copy(
        labels_hbm.at[pl.ds(g * K, K)], lbuf.at[b], sem_l[b])
    return cv, cl

  inflight = start(0)
  for g in range(NCHUNK):
    b = g % 2
    cv, cl = inflight
    cv.wait()
    cl.wait()
    if g + 1 < NCHUNK:
      inflight = start(g + 1)

    @plsc.parallel_loop(0, GPC, unroll=4)
    def _group(j):
      lab = lbuf[b, pl.ds(j * L, L)]
      idx0 = lab + lane_base
      vs = [vbuf[b, r, pl.ds(j * L, L)] for r in range(RPW)]
      idxs = [idx0 + (r * HIST) for r in range(RPW)]
      for r in range(RPW):
        plsc.addupdate_scatter(acc, [idxs[r]], vs[r])

    def _cnt(jj, _):
      lab = lbuf[b, pl.ds((sid * (GPC // NS) + jj) * L, L)]
      plsc.addupdate_scatter(cnt_acc, [lab + lane_base], ones)
      return 0
    lax.fori_loop(0, GPC // NS, _cnt, 0)

  def _redc(l, s):
    return s + cnt_acc[pl.ds(l * L, L)]
  cntb[...] = lax.fori_loop(0, L, _redc, zeros)
  pltpu.sync_copy(cntb, cnt_sh.at[sid])
  plsc.subcore_barrier()
  pltpu.sync_copy(cnt_sh, cnt_all)

  def _redall(t, s):
    return s + cnt_all[t, :]
  counts = lax.fori_loop(0, NS, _redall, zeros)

  present = counts > 0.5
  pos = plsc.cumsum(present.astype(jnp.int32)) - 1
  clb_v[...] = jnp.zeros((C,), jnp.int32)
  plsc.store_scatter(clb_v, [pos], iota, mask=present)
  clab = clb_v[...]

  for r in range(RPW):
    def _red(l, s):
      return s + acc[pl.ds(r * HIST + l * L, L)]
    rsbuf[...] = lax.fori_loop(0, L, _red, zeros)
    outb[r, :] = plsc.load_gather(rsbuf, [clab])
  pltpu.sync_copy(outb, out_hbm.at[pl.ds(out_base, RPW), :])


def _tc_body(lab_ref, val_ref, out_ref, acc_ref, cnt_ref):
  k = pl.program_id(0)

  @pl.when(k == 0)
  def _():
    acc_ref[...] = jnp.zeros_like(acc_ref)
    cnt_ref[...] = jnp.zeros_like(cnt_ref)

  lab = lab_ref[0, :]
  seg = lax.broadcasted_iota(jnp.int32, (K,), 0) // (K // NSEG)
  idx = lab + seg * C
  oh = (idx[:, None] == lax.broadcasted_iota(jnp.int32, (K, WID), 1)
        ).astype(jnp.float32)
  acc_ref[...] += jnp.dot(val_ref[...], oh,
                          preferred_element_type=jnp.float32)

  oht = (lab[None, :] == lax.broadcasted_iota(jnp.int32, (C, K), 0)
         ).astype(jnp.float32)
  cnt_ref[...] += jnp.sum(oht, axis=1, keepdims=True)

  @pl.when(k == NCHUNK - 1)
  def _():
    presb = cnt_ref[...] > 0.5
    crow = lax.broadcasted_iota(jnp.int32, (C, C), 0)
    ccol = lax.broadcasted_iota(jnp.int32, (C, C), 1)
    tril = (ccol <= crow).astype(jnp.float32)
    posc = jnp.dot(tril, presb.astype(jnp.float32),
                   preferred_element_type=jnp.float32) - 1.0
    pmain = jnp.logical_and(presb, posc.astype(jnp.int32) == ccol)
    colhas = jnp.sum(pmain.astype(jnp.float32), axis=0, keepdims=True)
    pfill = jnp.logical_and(crow == 0, colhas < 0.5)
    perm = jnp.logical_or(pmain, pfill).astype(jnp.float32)
    wrow = lax.broadcasted_iota(jnp.int32, (WID, C), 0)
    wcol = lax.broadcasted_iota(jnp.int32, (WID, C), 1)
    fold = (wrow % C == wcol).astype(jnp.float32)
    fp = jnp.dot(fold, perm, preferred_element_type=jnp.float32,
                 precision=lax.Precision.HIGHEST)
    out_ref[...] = jnp.dot(acc_ref[...], fp,
                           preferred_element_type=jnp.float32,
                           precision=lax.Precision.HIGHEST)


@jax.jit
def _pooled_sums(values, labels):
  sums_tc = pl.pallas_call(
      _tc_body,
      grid=(NCHUNK,),
      in_specs=[
          pl.BlockSpec((1, K), lambda k: (0, k)),
          pl.BlockSpec((R_TC, K), lambda k: (0, k)),
      ],
      out_specs=pl.BlockSpec((R_TC, C), lambda k: (0, 0)),
      out_shape=jax.ShapeDtypeStruct((R_TC, C), jnp.float32),
      scratch_shapes=[
          pltpu.VMEM((R_TC, WID), jnp.float32),
          pltpu.VMEM((C, 1), jnp.float32),
      ],
  )(labels.reshape(1, N), values)

  sums_sc = pl.kernel(
      _body,
      out_type=jax.ShapeDtypeStruct((R_SC, C), jnp.float32),
      mesh=plsc.VectorSubcoreMesh(
          core_axis_name="c", subcore_axis_name="s",
          num_cores=NC, num_subcores=NS),
      compiler_params=pltpu.CompilerParams(needs_layout_passes=False),
      scratch_types=[
          pltpu.VMEM((2, RPW, K), jnp.float32),
          pltpu.VMEM((2, K), jnp.int32),
          pltpu.VMEM((RPW * HIST,), jnp.float32),
          pltpu.VMEM((HIST,), jnp.float32),
          pltpu.VMEM((RPW, C), jnp.float32),
          pltpu.VMEM((C,), jnp.float32),
          pltpu.VMEM((NS, C), jnp.float32),
          pltpu.VMEM((C,), jnp.float32),
          pltpu.VMEM((C,), jnp.int32),
          pltpu.VMEM_SHARED((NS, C), jnp.float32),
          pltpu.SemaphoreType.DMA,
          pltpu.SemaphoreType.DMA,
          pltpu.SemaphoreType.DMA,
          pltpu.SemaphoreType.DMA,
      ],
  )(values, labels)

  return jnp.concatenate([sums_tc, sums_sc], axis=0)


def kernel(values, labels):
  return _pooled_sums(values, labels)


# speedup vs baseline: 1.6233x; 1.0632x over previous
import functools

import jax
import jax.numpy as jnp
from jax import lax
from jax.experimental import pallas as pl
from jax.experimental.pallas import tpu as pltpu
from jax.experimental.pallas import tpu_sc as plsc

B = 256
N = 32768
C = 16
L = 16
NC = 2
NS = 16
NW = NC * NS
R_SC = 32
RPW = R_SC // NW
K = 4096
NCHUNK = N // K
HIST = 256
GPC = K // L
R_TC = B - R_SC
NSEG = 8
WID = NSEG * C


def _body(values_hbm, labels_hbm, out_hbm,
          vbuf, lbuf, acc, cnt_acc, outb, cntb, cnt_all, rsbuf, clb_v,
          cnt_sh, sem_v0, sem_v1, sem_l0, sem_l1):
  cid = lax.axis_index("c")
  sid = lax.axis_index("s")
  wid = sid * NC + cid
  out_base = wid * RPW
  row_base = R_TC + out_base
  sem_v = (sem_v0, sem_v1)
  sem_l = (sem_l0, sem_l1)

  zeros = jnp.zeros((L,), jnp.float32)
  ones = jnp.ones((L,), jnp.float32)
  iota = lax.broadcasted_iota(jnp.int32, (L,), 0)
  lane_base = iota * C

  def _zero(i, _):
    acc[pl.ds(i * L, L)] = zeros
    return 0
  lax.fori_loop(0, (RPW * HIST) // L, _zero, 0)

  def _zero_cnt(i, _):
    cnt_acc[pl.ds(i * L, L)] = zeros
    return 0
  lax.fori_loop(0, HIST // L, _zero_cnt, 0)

  def start(g):
    b = g % 2
    cv = pltpu.async_copy(
        values_hbm.at[pl.ds(row_base, RPW), pl.ds(g * K, K)],
        vbuf.at[b], sem_v[b])
    cl = pltpu.async_copy(
        labels_hbm.at[pl.ds(g * K, K)], lbuf.at[b], sem_l[b])
    return cv, cl

  inflight = start(0)
  for g in range(NCHUNK):
    b = g % 2
    cv, cl = inflight
    cv.wait()
    cl.wait()
    if g + 1 < NCHUNK:
      inflight = start(g + 1)

    @plsc.parallel_loop(0, GPC, unroll=4)
    def _group(j):
      lab = lbuf[b, pl.ds(j * L, L)]
      idx0 = lab + lane_base
      vs = [vbuf[b, r, pl.ds(j * L, L)] for r in range(RPW)]
      idxs = [idx0 + (r * HIST) for r in range(RPW)]
      for r in range(RPW):
        plsc.addupdate_scatter(acc, [idxs[r]], vs[r])

    def _cnt(jj, _):
      lab = lbuf[b, pl.ds((sid * (GPC // NS) + jj) * L, L)]
      plsc.addupdate_scatter(cnt_acc, [lab + lane_base], ones)
      return 0
    lax.fori_loop(0, GPC // NS, _cnt, 0)

  def _redc(l, s):
    return s + cnt_acc[pl.ds(l * L, L)]
  cntb[...] = lax.fori_loop(0, L, _redc, zeros)
  pltpu.sync_copy(cntb, cnt_sh.at[sid])
  plsc.subcore_barrier()
  pltpu.sync_copy(cnt_sh, cnt_all)

  def _redall(t, s):
    return s + cnt_all[t, :]
  counts = lax.fori_loop(0, NS, _redall, zeros)

  present = counts > 0.5
  pos = plsc.cumsum(present.astype(jnp.int32)) - 1
  clb_v[...] = jnp.zeros((C,), jnp.int32)
  plsc.store_scatter(clb_v, [pos], iota, mask=present)
  clab = clb_v[...]

  for r in range(RPW):
    def _red(l, s):
      return s + acc[pl.ds(r * HIST + l * L, L)]
    rsbuf[...] = lax.fori_loop(0, L, _red, zeros)
    outb[r, :] = plsc.load_gather(rsbuf, [clab])
  pltpu.sync_copy(outb, out_hbm.at[pl.ds(out_base, RPW), :])


def _tc_body(lab_ref, val_ref, out_ref, acc_ref, cnt_ref):
  k = pl.program_id(0)

  @pl.when(k == 0)
  def _():
    acc_ref[...] = jnp.zeros_like(acc_ref)
    cnt_ref[...] = jnp.zeros_like(cnt_ref)

  lab = lab_ref[0, :]
  seg = lax.broadcasted_iota(jnp.int32, (K,), 0) // (K // NSEG)
  idx = lab + seg * C
  oh = (idx[:, None] == lax.broadcasted_iota(jnp.int32, (K, WID), 1)
        ).astype(jnp.float32)
  acc_ref[...] += jnp.dot(val_ref[...], oh,
                          preferred_element_type=jnp.float32)

  oht = (lab[None, :] == lax.broadcasted_iota(jnp.int32, (C, K), 0)
         ).astype(jnp.float32)
  cnt_ref[...] += jnp.sum(oht, axis=1, keepdims=True)

  @pl.when(k == NCHUNK - 1)
  def _():
    presb = cnt_ref[...] > 0.5
    crow = lax.broadcasted_iota(jnp.int32, (C, C), 0)
    ccol = lax.broadcasted_iota(jnp.int32, (C, C), 1)
    tril = (ccol <= crow).astype(jnp.float32)
    posc = jnp.dot(tril, presb.astype(jnp.float32),
                   preferred_element_type=jnp.float32) - 1.0
    pmain = jnp.logical_and(presb, posc.astype(jnp.int32) == ccol)
    colhas = jnp.sum(pmain.astype(jnp.float32), axis=0, keepdims=True)
    pfill = jnp.logical_and(crow == 0, colhas < 0.5)
    perm = jnp.logical_or(pmain, pfill).astype(jnp.float32)
    wrow = lax.broadcasted_iota(jnp.int32, (WID, C), 0)
    wcol = lax.broadcasted_iota(jnp.int32, (WID, C), 1)
    fold = (wrow % C == wcol).astype(jnp.float32)
    fp = jnp.dot(fold, perm, preferred_element_type=jnp.float32,
                 precision=lax.Precision.HIGHEST)
    out_ref[...] = jnp.dot(acc_ref[...], fp,
                           preferred_element_type=jnp.float32,
                           precision=lax.Precision.HIGHEST)


@jax.jit
def _pooled_sums(values, labels):
  sums_tc = pl.pallas_call(
      _tc_body,
      grid=(NCHUNK,),
      in_specs=[
          pl.BlockSpec((1, K), lambda k: (0, k)),
          pl.BlockSpec((R_TC, K), lambda k: (0, k)),
      ],
      out_specs=pl.BlockSpec((R_TC, C), lambda k: (0, 0)),
      out_shape=jax.ShapeDtypeStruct((R_TC, C), jnp.float32),
      scratch_shapes=[
          pltpu.VMEM((R_TC, WID), jnp.float32),
          pltpu.VMEM((C, 1), jnp.float32),
      ],
  )(labels.reshape(1, N), values)

  sums_sc = pl.kernel(
      _body,
      out_type=jax.ShapeDtypeStruct((R_SC, C), jnp.float32),
      mesh=plsc.VectorSubcoreMesh(
          core_axis_name="c", subcore_axis_name="s",
          num_cores=NC, num_subcores=NS),
      compiler_params=pltpu.CompilerParams(needs_layout_passes=False),
      scratch_types=[
          pltpu.VMEM((2, RPW, K), jnp.float32),
          pltpu.VMEM((2, K), jnp.int32),
          pltpu.VMEM((RPW * HIST,), jnp.float32),
          pltpu.VMEM((HIST,), jnp.float32),
          pltpu.VMEM((RPW, C), jnp.float32),
          pltpu.VMEM((C,), jnp.float32),
          pltpu.VMEM((NS, C), jnp.float32),
          pltpu.VMEM((C,), jnp.float32),
          pltpu.VMEM((C,), jnp.int32),
          pltpu.VMEM_SHARED((NS, C), jnp.float32),
          pltpu.SemaphoreType.DMA,
          pltpu.SemaphoreType.DMA,
          pltpu.SemaphoreType.DMA,
          pltpu.SemaphoreType.DMA,
      ],
  )(values, labels)

  return jnp.concatenate([sums_tc, sums_sc], axis=0)


def kernel(values, labels):
  return _pooled_sums(values, labels)


# speedup vs baseline: 1.6598x; 1.0225x over previous
import functools

import jax
import jax.numpy as jnp
from jax import lax
from jax.experimental import pallas as pl
from jax.experimental.pallas import tpu as pltpu
from jax.experimental.pallas import tpu_sc as plsc

B = 256
N = 32768
C = 16
L = 16
NC = 2
NS = 16
NW = NC * NS
R_SC = 32
RPW = R_SC // NW
K = 8192
NCHUNK = N // K
HIST = 256
GPC = K // L
R_TC = B - R_SC
NSEG = 8
WID = NSEG * C


def _body(values_hbm, labels_hbm, out_hbm,
          vbuf, lbuf, acc, cnt_acc, outb, cntb, cnt_all, rsbuf, clb_v,
          cnt_sh, sem_v0, sem_v1, sem_l0, sem_l1):
  cid = lax.axis_index("c")
  sid = lax.axis_index("s")
  wid = sid * NC + cid
  out_base = wid * RPW
  row_base = R_TC + out_base
  sem_v = (sem_v0, sem_v1)
  sem_l = (sem_l0, sem_l1)

  zeros = jnp.zeros((L,), jnp.float32)
  ones = jnp.ones((L,), jnp.float32)
  iota = lax.broadcasted_iota(jnp.int32, (L,), 0)
  lane_base = iota * C

  def _zero(i, _):
    acc[pl.ds(i * L, L)] = zeros
    return 0
  lax.fori_loop(0, (RPW * HIST) // L, _zero, 0)

  def _zero_cnt(i, _):
    cnt_acc[pl.ds(i * L, L)] = zeros
    return 0
  lax.fori_loop(0, HIST // L, _zero_cnt, 0)

  def start(g):
    b = g % 2
    cv = pltpu.async_copy(
        values_hbm.at[pl.ds(row_base, RPW), pl.ds(g * K, K)],
        vbuf.at[b], sem_v[b])
    cl = pltpu.async_copy(
        labels_hbm.at[pl.ds(g * K, K)], lbuf.at[b], sem_l[b])
    return cv, cl

  inflight = start(0)
  for g in range(NCHUNK):
    b = g % 2
    cv, cl = inflight
    cv.wait()
    cl.wait()
    if g + 1 < NCHUNK:
      inflight = start(g + 1)

    @plsc.parallel_loop(0, GPC, unroll=4)
    def _group(j):
      lab = lbuf[b, pl.ds(j * L, L)]
      idx0 = lab + lane_base
      vs = [vbuf[b, r, pl.ds(j * L, L)] for r in range(RPW)]
      idxs = [idx0 + (r * HIST) for r in range(RPW)]
      for r in range(RPW):
        plsc.addupdate_scatter(acc, [idxs[r]], vs[r])

    def _cnt(jj, _):
      lab = lbuf[b, pl.ds((sid * (GPC // NS) + jj) * L, L)]
      plsc.addupdate_scatter(cnt_acc, [lab + lane_base], ones)
      return 0
    lax.fori_loop(0, GPC // NS, _cnt, 0)

  def _redc(l, s):
    return s + cnt_acc[pl.ds(l * L, L)]
  cntb[...] = lax.fori_loop(0, L, _redc, zeros)
  pltpu.sync_copy(cntb, cnt_sh.at[sid])
  plsc.subcore_barrier()
  pltpu.sync_copy(cnt_sh, cnt_all)

  def _redall(t, s):
    return s + cnt_all[t, :]
  counts = lax.fori_loop(0, NS, _redall, zeros)

  present = counts > 0.5
  pos = plsc.cumsum(present.astype(jnp.int32)) - 1
  clb_v[...] = jnp.zeros((C,), jnp.int32)
  plsc.store_scatter(clb_v, [pos], iota, mask=present)
  clab = clb_v[...]

  for r in range(RPW):
    def _red(l, s):
      return s + acc[pl.ds(r * HIST + l * L, L)]
    rsbuf[...] = lax.fori_loop(0, L, _red, zeros)
    outb[r, :] = plsc.load_gather(rsbuf, [clab])
  pltpu.sync_copy(outb, out_hbm.at[pl.ds(out_base, RPW), :])


def _tc_body(lab_ref, val_ref, out_ref, acc_ref, cnt_ref):
  k = pl.program_id(0)

  @pl.when(k == 0)
  def _():
    acc_ref[...] = jnp.zeros_like(acc_ref)
    cnt_ref[...] = jnp.zeros_like(cnt_ref)

  lab = lab_ref[0, :]
  seg = lax.broadcasted_iota(jnp.int32, (K,), 0) // (K // NSEG)
  idx = lab + seg * C
  oh = (idx[:, None] == lax.broadcasted_iota(jnp.int32, (K, WID), 1)
        ).astype(jnp.float32)
  acc_ref[...] += jnp.dot(val_ref[...], oh,
                          preferred_element_type=jnp.float32)

  oht = (lab[None, :] == lax.broadcasted_iota(jnp.int32, (C, K), 0)
         ).astype(jnp.float32)
  cnt_ref[...] += jnp.sum(oht, axis=1, keepdims=True)

  @pl.when(k == NCHUNK - 1)
  def _():
    presb = cnt_ref[...] > 0.5
    crow = lax.broadcasted_iota(jnp.int32, (C, C), 0)
    ccol = lax.broadcasted_iota(jnp.int32, (C, C), 1)
    tril = (ccol <= crow).astype(jnp.float32)
    posc = jnp.dot(tril, presb.astype(jnp.float32),
                   preferred_element_type=jnp.float32) - 1.0
    pmain = jnp.logical_and(presb, posc.astype(jnp.int32) == ccol)
    colhas = jnp.sum(pmain.astype(jnp.float32), axis=0, keepdims=True)
    pfill = jnp.logical_and(crow == 0, colhas < 0.5)
    perm = jnp.logical_or(pmain, pfill).astype(jnp.float32)
    wrow = lax.broadcasted_iota(jnp.int32, (WID, C), 0)
    wcol = lax.broadcasted_iota(jnp.int32, (WID, C), 1)
    fold = (wrow % C == wcol).astype(jnp.float32)
    fp = jnp.dot(fold, perm, preferred_element_type=jnp.float32,
                 precision=lax.Precision.HIGHEST)
    out_ref[...] = jnp.dot(acc_ref[...], fp,
                           preferred_element_type=jnp.float32,
                           precision=lax.Precision.HIGHEST)


@jax.jit
def _pooled_sums(values, labels):
  sums_tc = pl.pallas_call(
      _tc_body,
      grid=(NCHUNK,),
      in_specs=[
          pl.BlockSpec((1, K), lambda k: (0, k)),
          pl.BlockSpec((R_TC, K), lambda k: (0, k)),
      ],
      out_specs=pl.BlockSpec((R_TC, C), lambda k: (0, 0)),
      out_shape=jax.ShapeDtypeStruct((R_TC, C), jnp.float32),
      scratch_shapes=[
          pltpu.VMEM((R_TC, WID), jnp.float32),
          pltpu.VMEM((C, 1), jnp.float32),
      ],
  )(labels.reshape(1, N), values)

  sums_sc = pl.kernel(
      _body,
      out_type=jax.ShapeDtypeStruct((R_SC, C), jnp.float32),
      mesh=plsc.VectorSubcoreMesh(
          core_axis_name="c", subcore_axis_name="s",
          num_cores=NC, num_subcores=NS),
      compiler_params=pltpu.CompilerParams(needs_layout_passes=False),
      scratch_types=[
          pltpu.VMEM((2, RPW, K), jnp.float32),
          pltpu.VMEM((2, K), jnp.int32),
          pltpu.VMEM((RPW * HIST,), jnp.float32),
          pltpu.VMEM((HIST,), jnp.float32),
          pltpu.VMEM((RPW, C), jnp.float32),
          pltpu.VMEM((C,), jnp.float32),
          pltpu.VMEM((NS, C), jnp.float32),
          pltpu.VMEM((C,), jnp.float32),
          pltpu.VMEM((C,), jnp.int32),
          pltpu.VMEM_SHARED((NS, C), jnp.float32),
          pltpu.SemaphoreType.DMA,
          pltpu.SemaphoreType.DMA,
          pltpu.SemaphoreType.DMA,
          pltpu.SemaphoreType.DMA,
      ],
  )(values, labels)

  return jnp.concatenate([sums_tc, sums_sc], axis=0)


def kernel(values, labels):
  return _pooled_sums(values, labels)
